# baseline (speedup 1.0000x reference)
import functools
import jax
import jax.numpy as jnp
from jax.experimental import pallas as pl
from jax.experimental.pallas import tpu as pltpu
from jax.experimental.pallas import tpu_sc as plsc

D_MODEL = 768
WINDOW = 64
SEQ = 2048
BLK = 256
NB = SEQ // BLK
SLAB = BLK + WINDOW
INV_SQRT = 1.0 / (64.0 ** 0.5)

C_SC = 288
C_TC = D_MODEL - C_SC

NCORE, NSUB, LANES = 2, 16, 16
NWORK = NCORE * NSUB
ROWS_W = SEQ // NWORK
KROWS = ROWS_W + WINDOW
NCHUNK = C_SC // LANES


def _proj_kernel(x_ref, wq_ref, wk_ref, wv_ref, q_ref, k_ref, v_ref):
    x = x_ref[...]
    q_ref[...] = jnp.dot(x, wq_ref[...], preferred_element_type=jnp.float32)
    k_ref[...] = jnp.dot(x, wk_ref[...], preferred_element_type=jnp.float32)
    v_ref[...] = jnp.dot(x, wv_ref[...], preferred_element_type=jnp.float32)


def _win_sc_kernel(qsc_hbm, ksc_hbm, vsc_hbm, attn_hbm, q_t, k_t, v_t):
    cid = jax.lax.axis_index("c")
    sid = jax.lax.axis_index("s")
    wid = cid * NSUB + sid
    row0 = wid * ROWS_W

    pltpu.sync_copy(qsc_hbm.at[pl.ds(row0, ROWS_W)], q_t)

    @pl.when(wid == 0)
    def _stage_first():
        zero = jnp.zeros((LANES,), jnp.float32)

        def zbody(j, carry):
            for c in range(NCHUNK):
                k_t[j, pl.ds(c * LANES, LANES)] = zero
                v_t[j, pl.ds(c * LANES, LANES)] = zero
            return carry
        jax.lax.fori_loop(0, WINDOW, zbody, 0)
        pltpu.sync_copy(ksc_hbm.at[pl.ds(0, ROWS_W)],
                        k_t.at[pl.ds(WINDOW, ROWS_W)])
        pltpu.sync_copy(vsc_hbm.at[pl.ds(0, ROWS_W)],
                        v_t.at[pl.ds(WINDOW, ROWS_W)])

    @pl.when(wid > 0)
    def _stage_interior():
        pltpu.sync_copy(ksc_hbm.at[pl.ds(row0 - WINDOW, KROWS)], k_t)
        pltpu.sync_copy(vsc_hbm.at[pl.ds(row0 - WINDOW, KROWS)], v_t)

    GRP = 4

    def cbody(c, carry):
        col = c * LANES

        def mbody(j8, m):
            j = j8 * 8
            for u in range(8):
                m = jnp.maximum(m, jnp.abs(k_t[j + u, pl.ds(col, LANES)]))
            return m
        mcol = jax.lax.fori_loop(0, KROWS // 8, mbody,
                                 jnp.zeros((LANES,), jnp.float32))

        def gbody(g, carry2):
            s0 = g * GRP
            qs, ms, dens, accs = [], [], [], []
            for ds in range(GRP):
                q16 = q_t[s0 + ds, pl.ds(col, LANES)] * INV_SQRT
                m16 = jnp.abs(q16) * mcol
                ninv = jnp.maximum(WINDOW - 1 - (row0 + s0 + ds),
                                   0).astype(jnp.float32)
                qs.append(q16)
                ms.append(m16)
                dens.append(jnp.exp(-m16) * (-ninv))
                accs.append(jnp.zeros((LANES,), jnp.float32))
            for t in range(WINDOW + GRP - 1):
                kj = k_t[s0 + t + 1, pl.ds(col, LANES)]
                vj = v_t[s0 + t + 1, pl.ds(col, LANES)]
                for ds in range(max(0, t - WINDOW + 1), min(GRP - 1, t) + 1):
                    e = jnp.exp(qs[ds] * kj - ms[ds])
                    dens[ds] = dens[ds] + e
                    accs[ds] = accs[ds] + e * vj
            for ds in range(GRP):
                q_t[s0 + ds, pl.ds(col, LANES)] = accs[ds] / dens[ds]
            return carry2

        return jax.lax.fori_loop(0, ROWS_W // GRP, gbody, carry)

    jax.lax.fori_loop(0, NCHUNK, cbody, 0)

    pltpu.sync_copy(q_t, attn_hbm.at[pl.ds(row0, ROWS_W)])


def _win_tc_kernel(qt_ref, kt_ref, vt_ref, wo_ref, out_ref,
                   slab_k_ref, slab_v_ref, shk, shv, attn_ref):
    i = pl.program_id(0)
    base = pl.multiple_of(i * BLK, BLK)

    @pl.when(i == 0)
    def _first_block_slab():
        slab_k_ref[0:WINDOW, :] = jnp.zeros((WINDOW, C_TC), jnp.float32)
        slab_v_ref[0:WINDOW, :] = jnp.zeros((WINDOW, C_TC), jnp.float32)
        slab_k_ref[pl.ds(WINDOW, BLK), :] = kt_ref[pl.ds(0, BLK), :]
        slab_v_ref[pl.ds(WINDOW, BLK), :] = vt_ref[pl.ds(0, BLK), :]

    @pl.when(i > 0)
    def _interior_slab():
        slab_k_ref[0:SLAB, :] = kt_ref[pl.ds(base - WINDOW, SLAB), :]
        slab_v_ref[0:SLAB, :] = vt_ref[pl.ds(base - WINDOW, SLAB), :]

    slab_k = slab_k_ref[...]
    slab_v = slab_v_ref[...]
    for r in range(8):
        shk[r] = jax.lax.slice(slab_k, (r, 0), (r + SLAB, C_TC))
        shv[r] = jax.lax.slice(slab_v, (r, 0), (r + SLAB, C_TC))

    colmax = jnp.max(
        jnp.abs(jax.lax.slice(slab_k, (0, 0), (SLAB, C_TC))),
        axis=0, keepdims=True)

    TR = 8

    def tile_body(t, carry):
        t8 = pl.multiple_of(t * TR, TR)
        q_scaled = qt_ref[pl.ds(t8, TR), :] * INV_SQRT
        m = jnp.abs(q_scaled) * colmax
        rows = jax.lax.broadcasted_iota(jnp.int32, (TR, C_TC), 0) + base + t8
        n_invalid = jnp.maximum(WINDOW - 1 - rows, 0).astype(jnp.float32)
        den = -n_invalid * jnp.exp(-m)
        acc = jnp.zeros((TR, C_TC), jnp.float32)
        for w in range(WINDOW):
            s0 = w + 1
            r = s0 % 8
            a = pl.multiple_of(s0 - r + t8, TR)
            e = jnp.exp(q_scaled * shk[r, pl.ds(a, TR), :] - m)
            den = den + e
            acc = acc + e * shv[r, pl.ds(a, TR), :]
        attn_ref[pl.ds(t8, TR), :] = acc / den
        return carry

    jax.lax.fori_loop(0, BLK // TR, tile_body, 0)

    out_ref[...] = jnp.dot(attn_ref[...], wo_ref[...],
                           preferred_element_type=jnp.float32)


def _combine_kernel(attnsc_ref, wosc_ref, part_ref, out_ref):
    out_ref[...] = part_ref[...] + jnp.dot(
        attnsc_ref[...], wosc_ref[...], preferred_element_type=jnp.float32)


def kernel(x, Wq, Wk, Wv, Wo):
    xs = x.reshape(SEQ, D_MODEL)
    wqt, wkt, wvt, wot = Wq.T, Wk.T, Wv.T, Wo.T
    wot_sc = wot[:C_SC, :]
    wot_tc = wot[C_SC:, :]

    sc_spec = pl.BlockSpec((BLK, C_SC), lambda i: (i, 0))
    tc_spec = pl.BlockSpec((BLK, C_TC), lambda i: (i, 0))

    def proj(wq_cols, wk_cols, wv_cols, width, spec):
        return pl.pallas_call(
            _proj_kernel,
            grid=(NB,),
            in_specs=[
                pl.BlockSpec((BLK, D_MODEL), lambda i: (i, 0)),
                pl.BlockSpec((D_MODEL, width), lambda i: (0, 0)),
                pl.BlockSpec((D_MODEL, width), lambda i: (0, 0)),
                pl.BlockSpec((D_MODEL, width), lambda i: (0, 0)),
            ],
            out_specs=[spec, spec, spec],
            out_shape=[jax.ShapeDtypeStruct((SEQ, width), jnp.float32)] * 3,
        )(xs, wq_cols, wk_cols, wv_cols)

    qsc, ksc, vsc = proj(wqt[:, :C_SC], wkt[:, :C_SC], wvt[:, :C_SC],
                         C_SC, sc_spec)
    qtc, ktc, vtc = proj(wqt[:, C_SC:], wkt[:, C_SC:], wvt[:, C_SC:],
                         C_TC, tc_spec)

    attn_sc = pl.kernel(
        _win_sc_kernel,
        out_type=jax.ShapeDtypeStruct((SEQ, C_SC), jnp.float32),
        mesh=plsc.VectorSubcoreMesh(
            core_axis_name="c", subcore_axis_name="s",
            num_cores=NCORE, num_subcores=NSUB),
        scratch_types=[
            pltpu.VMEM((ROWS_W, C_SC), jnp.float32),
            pltpu.VMEM((KROWS, C_SC), jnp.float32),
            pltpu.VMEM((KROWS, C_SC), jnp.float32),
        ],
    )(qsc, ksc, vsc)

    full_tc = pl.BlockSpec((SEQ, C_TC), lambda i: (0, 0))
    out_part = pl.pallas_call(
        _win_tc_kernel,
        grid=(NB,),
        in_specs=[
            tc_spec,
            full_tc,
            full_tc,
            pl.BlockSpec((C_TC, D_MODEL), lambda i: (0, 0)),
        ],
        out_specs=pl.BlockSpec((BLK, D_MODEL), lambda i: (i, 0)),
        out_shape=jax.ShapeDtypeStruct((SEQ, D_MODEL), jnp.float32),
        scratch_shapes=[
            pltpu.VMEM((SLAB + 8, C_TC), jnp.float32),
            pltpu.VMEM((SLAB + 8, C_TC), jnp.float32),
            pltpu.VMEM((8, SLAB, C_TC), jnp.float32),
            pltpu.VMEM((8, SLAB, C_TC), jnp.float32),
            pltpu.VMEM((BLK, C_TC), jnp.float32),
        ],
    )(qtc, ktc, vtc, wot_tc)

    out = pl.pallas_call(
        _combine_kernel,
        grid=(NB,),
        in_specs=[
            sc_spec,
            pl.BlockSpec((C_SC, D_MODEL), lambda i: (0, 0)),
            pl.BlockSpec((BLK, D_MODEL), lambda i: (i, 0)),
        ],
        out_specs=pl.BlockSpec((BLK, D_MODEL), lambda i: (i, 0)),
        out_shape=jax.ShapeDtypeStruct((SEQ, D_MODEL), jnp.float32),
    )(attn_sc, wot_sc, out_part)

    return out.reshape(1, SEQ, D_MODEL)


# speedup vs baseline: 1.1103x; 1.1103x over previous
import functools
import jax
import jax.numpy as jnp
from jax.experimental import pallas as pl
from jax.experimental.pallas import tpu as pltpu
from jax.experimental.pallas import tpu_sc as plsc

D_MODEL = 768
WINDOW = 64
SEQ = 2048
BLK = 256
NB = SEQ // BLK
SLAB = BLK + WINDOW
INV_SQRT = 1.0 / (64.0 ** 0.5)

C_SC = 256
C_TC = D_MODEL - C_SC

NCORE, NSUB, LANES = 2, 16, 16
NWORK = NCORE * NSUB
ROWS_W = SEQ // NWORK
KROWS = ROWS_W + WINDOW
NCHUNK = C_SC // LANES


def _proj_kernel(x_ref, wq_ref, wk_ref, wv_ref, q_ref, k_ref, v_ref):
    x = x_ref[...]
    q_ref[...] = jnp.dot(x, wq_ref[...], preferred_element_type=jnp.float32)
    k_ref[...] = jnp.dot(x, wk_ref[...], preferred_element_type=jnp.float32)
    v_ref[...] = jnp.dot(x, wv_ref[...], preferred_element_type=jnp.float32)


def _win_sc_kernel(qsc_hbm, ksc_hbm, vsc_hbm, attn_hbm, q_t, k_t, v_t):
    cid = jax.lax.axis_index("c")
    sid = jax.lax.axis_index("s")
    wid = cid * NSUB + sid
    row0 = wid * ROWS_W

    pltpu.sync_copy(qsc_hbm.at[pl.ds(row0, ROWS_W)], q_t)

    @pl.when(wid == 0)
    def _stage_first():
        zero = jnp.zeros((LANES,), jnp.float32)

        def zbody(j, carry):
            for c in range(NCHUNK):
                k_t[j, pl.ds(c * LANES, LANES)] = zero
                v_t[j, pl.ds(c * LANES, LANES)] = zero
            return carry
        jax.lax.fori_loop(0, WINDOW, zbody, 0)
        pltpu.sync_copy(ksc_hbm.at[pl.ds(0, ROWS_W)],
                        k_t.at[pl.ds(WINDOW, ROWS_W)])
        pltpu.sync_copy(vsc_hbm.at[pl.ds(0, ROWS_W)],
                        v_t.at[pl.ds(WINDOW, ROWS_W)])

    @pl.when(wid > 0)
    def _stage_interior():
        pltpu.sync_copy(ksc_hbm.at[pl.ds(row0 - WINDOW, KROWS)], k_t)
        pltpu.sync_copy(vsc_hbm.at[pl.ds(row0 - WINDOW, KROWS)], v_t)

    GRP = 4

    def cbody(c, carry):
        col = c * LANES

        def mbody(j8, m):
            j = j8 * 8
            for u in range(8):
                m = jnp.maximum(m, jnp.abs(k_t[j + u, pl.ds(col, LANES)]))
            return m
        mcol = jax.lax.fori_loop(0, KROWS // 8, mbody,
                                 jnp.zeros((LANES,), jnp.float32))

        def gbody(g, carry2):
            s0 = g * GRP
            qs, ms, dens, accs = [], [], [], []
            for ds in range(GRP):
                q16 = q_t[s0 + ds, pl.ds(col, LANES)] * INV_SQRT
                m16 = jnp.abs(q16) * mcol
                ninv = jnp.maximum(WINDOW - 1 - (row0 + s0 + ds),
                                   0).astype(jnp.float32)
                qs.append(q16)
                ms.append(m16)
                dens.append(jnp.exp(-m16) * (-ninv))
                accs.append(jnp.zeros((LANES,), jnp.float32))
            for t in range(WINDOW + GRP - 1):
                kj = k_t[s0 + t + 1, pl.ds(col, LANES)]
                vj = v_t[s0 + t + 1, pl.ds(col, LANES)]
                for ds in range(max(0, t - WINDOW + 1), min(GRP - 1, t) + 1):
                    e = jnp.exp(qs[ds] * kj - ms[ds])
                    dens[ds] = dens[ds] + e
                    accs[ds] = accs[ds] + e * vj
            for ds in range(GRP):
                q_t[s0 + ds, pl.ds(col, LANES)] = accs[ds] / dens[ds]
            return carry2

        return jax.lax.fori_loop(0, ROWS_W // GRP, gbody, carry)

    jax.lax.fori_loop(0, NCHUNK, cbody, 0)

    pltpu.sync_copy(q_t, attn_hbm.at[pl.ds(row0, ROWS_W)])


def _win_tc_kernel(qt_ref, kt_ref, vt_ref, wo_ref, out_ref,
                   slab_k_ref, slab_v_ref, shk, shv, attn_ref):
    i = pl.program_id(0)
    base = pl.multiple_of(i * BLK, BLK)

    @pl.when(i == 0)
    def _first_block_slab():
        slab_k_ref[0:WINDOW, :] = jnp.zeros((WINDOW, C_TC), jnp.float32)
        slab_v_ref[0:WINDOW, :] = jnp.zeros((WINDOW, C_TC), jnp.float32)
        slab_k_ref[pl.ds(WINDOW, BLK), :] = kt_ref[pl.ds(0, BLK), :]
        slab_v_ref[pl.ds(WINDOW, BLK), :] = vt_ref[pl.ds(0, BLK), :]

    @pl.when(i > 0)
    def _interior_slab():
        slab_k_ref[0:SLAB, :] = kt_ref[pl.ds(base - WINDOW, SLAB), :]
        slab_v_ref[0:SLAB, :] = vt_ref[pl.ds(base - WINDOW, SLAB), :]

    slab_k = slab_k_ref[...]
    slab_v = slab_v_ref[...]
    for r in range(8):
        shk[r] = jax.lax.slice(slab_k, (r, 0), (r + SLAB, C_TC))
        shv[r] = jax.lax.slice(slab_v, (r, 0), (r + SLAB, C_TC))

    colmax = jnp.max(
        jnp.abs(jax.lax.slice(slab_k, (0, 0), (SLAB, C_TC))),
        axis=0, keepdims=True)

    TR = 8

    def tile_body(t, carry):
        t8 = pl.multiple_of(t * TR, TR)
        q_scaled = qt_ref[pl.ds(t8, TR), :] * INV_SQRT
        m = jnp.abs(q_scaled) * colmax
        rows = jax.lax.broadcasted_iota(jnp.int32, (TR, C_TC), 0) + base + t8
        n_invalid = jnp.maximum(WINDOW - 1 - rows, 0).astype(jnp.float32)
        den = -n_invalid * jnp.exp(-m)
        acc = jnp.zeros((TR, C_TC), jnp.float32)
        for w in range(WINDOW):
            s0 = w + 1
            r = s0 % 8
            a = pl.multiple_of(s0 - r + t8, TR)
            e = jnp.exp(q_scaled * shk[r, pl.ds(a, TR), :] - m)
            den = den + e
            acc = acc + e * shv[r, pl.ds(a, TR), :]
        attn_ref[pl.ds(t8, TR), :] = acc / den
        return carry

    jax.lax.fori_loop(0, BLK // TR, tile_body, 0)

    out_ref[...] = jnp.dot(attn_ref[...], wo_ref[...],
                           preferred_element_type=jnp.float32)


def _combine_kernel(attnsc_ref, wosc_ref, part_ref, out_ref):
    out_ref[...] = part_ref[...] + jnp.dot(
        attnsc_ref[...], wosc_ref[...], preferred_element_type=jnp.float32)


def kernel(x, Wq, Wk, Wv, Wo):
    xs = x.reshape(SEQ, D_MODEL)
    wqt, wkt, wvt, wot = Wq.T, Wk.T, Wv.T, Wo.T
    wot_sc = wot[:C_SC, :]
    wot_tc = wot[C_SC:, :]

    sc_spec = pl.BlockSpec((BLK, C_SC), lambda i: (i, 0))
    tc_spec = pl.BlockSpec((BLK, C_TC), lambda i: (i, 0))

    def proj(wq_cols, wk_cols, wv_cols, width, spec):
        return pl.pallas_call(
            _proj_kernel,
            grid=(NB,),
            in_specs=[
                pl.BlockSpec((BLK, D_MODEL), lambda i: (i, 0)),
                pl.BlockSpec((D_MODEL, width), lambda i: (0, 0)),
                pl.BlockSpec((D_MODEL, width), lambda i: (0, 0)),
                pl.BlockSpec((D_MODEL, width), lambda i: (0, 0)),
            ],
            out_specs=[spec, spec, spec],
            out_shape=[jax.ShapeDtypeStruct((SEQ, width), jnp.float32)] * 3,
        )(xs, wq_cols, wk_cols, wv_cols)

    qsc, ksc, vsc = proj(wqt[:, :C_SC], wkt[:, :C_SC], wvt[:, :C_SC],
                         C_SC, sc_spec)
    qtc, ktc, vtc = proj(wqt[:, C_SC:], wkt[:, C_SC:], wvt[:, C_SC:],
                         C_TC, tc_spec)

    attn_sc = pl.kernel(
        _win_sc_kernel,
        out_type=jax.ShapeDtypeStruct((SEQ, C_SC), jnp.float32),
        mesh=plsc.VectorSubcoreMesh(
            core_axis_name="c", subcore_axis_name="s",
            num_cores=NCORE, num_subcores=NSUB),
        scratch_types=[
            pltpu.VMEM((ROWS_W, C_SC), jnp.float32),
            pltpu.VMEM((KROWS, C_SC), jnp.float32),
            pltpu.VMEM((KROWS, C_SC), jnp.float32),
        ],
    )(qsc, ksc, vsc)

    full_tc = pl.BlockSpec((SEQ, C_TC), lambda i: (0, 0))
    out_part = pl.pallas_call(
        _win_tc_kernel,
        grid=(NB,),
        in_specs=[
            tc_spec,
            full_tc,
            full_tc,
            pl.BlockSpec((C_TC, D_MODEL), lambda i: (0, 0)),
        ],
        out_specs=pl.BlockSpec((BLK, D_MODEL), lambda i: (i, 0)),
        out_shape=jax.ShapeDtypeStruct((SEQ, D_MODEL), jnp.float32),
        scratch_shapes=[
            pltpu.VMEM((SLAB + 8, C_TC), jnp.float32),
            pltpu.VMEM((SLAB + 8, C_TC), jnp.float32),
            pltpu.VMEM((8, SLAB, C_TC), jnp.float32),
            pltpu.VMEM((8, SLAB, C_TC), jnp.float32),
            pltpu.VMEM((BLK, C_TC), jnp.float32),
        ],
    )(qtc, ktc, vtc, wot_tc)

    out = pl.pallas_call(
        _combine_kernel,
        grid=(NB,),
        in_specs=[
            sc_spec,
            pl.BlockSpec((C_SC, D_MODEL), lambda i: (0, 0)),
            pl.BlockSpec((BLK, D_MODEL), lambda i: (i, 0)),
        ],
        out_specs=pl.BlockSpec((BLK, D_MODEL), lambda i: (i, 0)),
        out_shape=jax.ShapeDtypeStruct((SEQ, D_MODEL), jnp.float32),
    )(attn_sc, wot_sc, out_part)

    return out.reshape(1, SEQ, D_MODEL)


# speedup vs baseline: 1.1329x; 1.0203x over previous
import functools
import jax
import jax.numpy as jnp
from jax.experimental import pallas as pl
from jax.experimental.pallas import tpu as pltpu
from jax.experimental.pallas import tpu_sc as plsc

D_MODEL = 768
WINDOW = 64
SEQ = 2048
BLK = 256
NB = SEQ // BLK
SLAB = BLK + WINDOW
INV_SQRT = 1.0 / (64.0 ** 0.5)

C_SC = 256
C_TC = D_MODEL - C_SC

NCORE, NSUB, LANES = 2, 16, 16
NWORK = NCORE * NSUB
ROWS_W = SEQ // NWORK
KROWS = ROWS_W + WINDOW
NCHUNK = C_SC // LANES


def _proj_kernel(x_ref, wq_ref, wk_ref, wv_ref, q_ref, k_ref, v_ref):
    x = x_ref[...]
    q_ref[...] = jnp.dot(x, wq_ref[...], preferred_element_type=jnp.float32)
    k_ref[...] = jnp.dot(x, wk_ref[...], preferred_element_type=jnp.float32)
    v_ref[...] = jnp.dot(x, wv_ref[...], preferred_element_type=jnp.float32)


def _win_sc_kernel(qsc_hbm, ksc_hbm, vsc_hbm, attn_hbm, q_t, k_t, v_t):
    cid = jax.lax.axis_index("c")
    sid = jax.lax.axis_index("s")
    wid = cid * NSUB + sid
    row0 = wid * ROWS_W

    pltpu.sync_copy(qsc_hbm.at[pl.ds(row0, ROWS_W)], q_t)

    @pl.when(wid == 0)
    def _stage_first():
        zero = jnp.zeros((LANES,), jnp.float32)

        def zbody(j, carry):
            for c in range(NCHUNK):
                k_t[j, pl.ds(c * LANES, LANES)] = zero
                v_t[j, pl.ds(c * LANES, LANES)] = zero
            return carry
        jax.lax.fori_loop(0, WINDOW, zbody, 0)
        pltpu.sync_copy(ksc_hbm.at[pl.ds(0, ROWS_W)],
                        k_t.at[pl.ds(WINDOW, ROWS_W)])
        pltpu.sync_copy(vsc_hbm.at[pl.ds(0, ROWS_W)],
                        v_t.at[pl.ds(WINDOW, ROWS_W)])

    @pl.when(wid > 0)
    def _stage_interior():
        pltpu.sync_copy(ksc_hbm.at[pl.ds(row0 - WINDOW, KROWS)], k_t)
        pltpu.sync_copy(vsc_hbm.at[pl.ds(row0 - WINDOW, KROWS)], v_t)

    GRP = 4

    def cbody(c, carry):
        col = c * LANES

        def mbody(j8, m):
            j = j8 * 8
            for u in range(8):
                m = jnp.maximum(m, jnp.abs(k_t[j + u, pl.ds(col, LANES)]))
            return m
        mcol = jax.lax.fori_loop(0, KROWS // 8, mbody,
                                 jnp.zeros((LANES,), jnp.float32))

        def gbody(g, carry2):
            s0 = g * GRP
            qs, ms, dens, accs = [], [], [], []
            for ds in range(GRP):
                q16 = q_t[s0 + ds, pl.ds(col, LANES)] * INV_SQRT
                m16 = jnp.abs(q16) * mcol
                ninv = jnp.maximum(WINDOW - 1 - (row0 + s0 + ds),
                                   0).astype(jnp.float32)
                qs.append(q16)
                ms.append(m16)
                dens.append(jnp.exp(-m16) * (-ninv))
                accs.append(jnp.zeros((LANES,), jnp.float32))
            for t in range(WINDOW + GRP - 1):
                kj = k_t[s0 + t + 1, pl.ds(col, LANES)]
                vj = v_t[s0 + t + 1, pl.ds(col, LANES)]
                for ds in range(max(0, t - WINDOW + 1), min(GRP - 1, t) + 1):
                    e = jnp.exp(qs[ds] * kj - ms[ds])
                    dens[ds] = dens[ds] + e
                    accs[ds] = accs[ds] + e * vj
            for ds in range(GRP):
                q_t[s0 + ds, pl.ds(col, LANES)] = accs[ds] / dens[ds]
            return carry2

        return jax.lax.fori_loop(0, ROWS_W // GRP, gbody, carry)

    jax.lax.fori_loop(0, NCHUNK, cbody, 0)

    pltpu.sync_copy(q_t, attn_hbm.at[pl.ds(row0, ROWS_W)])


def _win_tc_kernel(qt_ref, kt_ref, vt_ref, wo_ref, out_ref,
                   slab_k_ref, slab_v_ref, shk, shv, attn_ref):
    i = pl.program_id(0)
    base = pl.multiple_of(i * BLK, BLK)

    @pl.when(i == 0)
    def _first_block_slab():
        slab_k_ref[0:WINDOW, :] = jnp.zeros((WINDOW, C_TC), jnp.float32)
        slab_v_ref[0:WINDOW, :] = jnp.zeros((WINDOW, C_TC), jnp.float32)
        slab_k_ref[pl.ds(WINDOW, BLK), :] = kt_ref[pl.ds(0, BLK), :]
        slab_v_ref[pl.ds(WINDOW, BLK), :] = vt_ref[pl.ds(0, BLK), :]

    @pl.when(i > 0)
    def _interior_slab():
        slab_k_ref[0:SLAB, :] = kt_ref[pl.ds(base - WINDOW, SLAB), :]
        slab_v_ref[0:SLAB, :] = vt_ref[pl.ds(base - WINDOW, SLAB), :]

    slab_k = slab_k_ref[...]
    slab_v = slab_v_ref[...]
    for r in range(8):
        shk[r] = jax.lax.slice(slab_k, (r, 0), (r + SLAB, C_TC))
        shv[r] = jax.lax.slice(slab_v, (r, 0), (r + SLAB, C_TC))

    colmax = jnp.max(
        jnp.abs(jax.lax.slice(slab_k, (0, 0), (SLAB, C_TC))),
        axis=0, keepdims=True)

    TR = 8

    def tile_body(t, carry):
        t8 = pl.multiple_of(t * TR, TR)
        q_scaled = qt_ref[pl.ds(t8, TR), :] * INV_SQRT
        m = jnp.abs(q_scaled) * colmax
        rows = jax.lax.broadcasted_iota(jnp.int32, (TR, C_TC), 0) + base + t8
        n_invalid = jnp.maximum(WINDOW - 1 - rows, 0).astype(jnp.float32)
        den = -n_invalid * jnp.exp(-m)
        acc = jnp.zeros((TR, C_TC), jnp.float32)
        for w in range(WINDOW):
            s0 = w + 1
            r = s0 % 8
            a = pl.multiple_of(s0 - r + t8, TR)
            e = jnp.exp(q_scaled * shk[r, pl.ds(a, TR), :] - m)
            den = den + e
            acc = acc + e * shv[r, pl.ds(a, TR), :]
        attn_ref[pl.ds(t8, TR), :] = acc / den
        return carry

    jax.lax.fori_loop(0, BLK // TR, tile_body, 0)

    out_ref[...] = jnp.dot(attn_ref[...].astype(jnp.bfloat16), wo_ref[...],
                           preferred_element_type=jnp.float32)


def _combine_kernel(attnsc_ref, wosc_ref, part_ref, out_ref):
    out_ref[...] = part_ref[...] + jnp.dot(
        attnsc_ref[...].astype(jnp.bfloat16), wosc_ref[...],
        preferred_element_type=jnp.float32)


def kernel(x, Wq, Wk, Wv, Wo):
    xs = x.reshape(SEQ, D_MODEL).astype(jnp.bfloat16)
    bf = jnp.bfloat16
    wqt, wkt, wvt, wot = Wq.T.astype(bf), Wk.T.astype(bf), Wv.T.astype(bf), Wo.T.astype(bf)
    wot_sc = wot[:C_SC, :]
    wot_tc = wot[C_SC:, :]

    sc_spec = pl.BlockSpec((BLK, C_SC), lambda i: (i, 0))
    tc_spec = pl.BlockSpec((BLK, C_TC), lambda i: (i, 0))

    def proj(wq_cols, wk_cols, wv_cols, width, spec):
        return pl.pallas_call(
            _proj_kernel,
            grid=(NB,),
            in_specs=[
                pl.BlockSpec((BLK, D_MODEL), lambda i: (i, 0)),
                pl.BlockSpec((D_MODEL, width), lambda i: (0, 0)),
                pl.BlockSpec((D_MODEL, width), lambda i: (0, 0)),
                pl.BlockSpec((D_MODEL, width), lambda i: (0, 0)),
            ],
            out_specs=[spec, spec, spec],
            out_shape=[jax.ShapeDtypeStruct((SEQ, width), jnp.float32)] * 3,
        )(xs, wq_cols, wk_cols, wv_cols)

    qsc, ksc, vsc = proj(wqt[:, :C_SC], wkt[:, :C_SC], wvt[:, :C_SC],
                         C_SC, sc_spec)
    qtc, ktc, vtc = proj(wqt[:, C_SC:], wkt[:, C_SC:], wvt[:, C_SC:],
                         C_TC, tc_spec)

    attn_sc = pl.kernel(
        _win_sc_kernel,
        out_type=jax.ShapeDtypeStruct((SEQ, C_SC), jnp.float32),
        mesh=plsc.VectorSubcoreMesh(
            core_axis_name="c", subcore_axis_name="s",
            num_cores=NCORE, num_subcores=NSUB),
        scratch_types=[
            pltpu.VMEM((ROWS_W, C_SC), jnp.float32),
            pltpu.VMEM((KROWS, C_SC), jnp.float32),
            pltpu.VMEM((KROWS, C_SC), jnp.float32),
        ],
    )(qsc, ksc, vsc)

    full_tc = pl.BlockSpec((SEQ, C_TC), lambda i: (0, 0))
    out_part = pl.pallas_call(
        _win_tc_kernel,
        grid=(NB,),
        in_specs=[
            tc_spec,
            full_tc,
            full_tc,
            pl.BlockSpec((C_TC, D_MODEL), lambda i: (0, 0)),
        ],
        out_specs=pl.BlockSpec((BLK, D_MODEL), lambda i: (i, 0)),
        out_shape=jax.ShapeDtypeStruct((SEQ, D_MODEL), jnp.float32),
        scratch_shapes=[
            pltpu.VMEM((SLAB + 8, C_TC), jnp.float32),
            pltpu.VMEM((SLAB + 8, C_TC), jnp.float32),
            pltpu.VMEM((8, SLAB, C_TC), jnp.float32),
            pltpu.VMEM((8, SLAB, C_TC), jnp.float32),
            pltpu.VMEM((BLK, C_TC), jnp.float32),
        ],
    )(qtc, ktc, vtc, wot_tc)

    out = pl.pallas_call(
        _combine_kernel,
        grid=(NB,),
        in_specs=[
            sc_spec,
            pl.BlockSpec((C_SC, D_MODEL), lambda i: (0, 0)),
            pl.BlockSpec((BLK, D_MODEL), lambda i: (i, 0)),
        ],
        out_specs=pl.BlockSpec((BLK, D_MODEL), lambda i: (i, 0)),
        out_shape=jax.ShapeDtypeStruct((SEQ, D_MODEL), jnp.float32),
    )(attn_sc, wot_sc, out_part)

    return out.reshape(1, SEQ, D_MODEL)


# speedup vs baseline: 1.1347x; 1.0016x over previous
import functools
import jax
import jax.numpy as jnp
from jax.experimental import pallas as pl
from jax.experimental.pallas import tpu as pltpu
from jax.experimental.pallas import tpu_sc as plsc

D_MODEL = 768
WINDOW = 64
SEQ = 2048
BLK = 256
NB = SEQ // BLK
SLAB = BLK + WINDOW
INV_SQRT = 1.0 / (64.0 ** 0.5)

C_SC = 256
C_TC = D_MODEL - C_SC

NCORE, NSUB, LANES = 2, 16, 16
NWORK = NCORE * NSUB
ROWS_W = SEQ // NWORK
KROWS = ROWS_W + WINDOW
NCHUNK = C_SC // LANES


def _proj_kernel(x_ref, wq_ref, wk_ref, wv_ref, q_ref, k_ref, v_ref):
    x = x_ref[...]
    q_ref[...] = jnp.dot(x, wq_ref[...], preferred_element_type=jnp.float32)
    k_ref[...] = jnp.dot(x, wk_ref[...], preferred_element_type=jnp.float32)
    v_ref[...] = jnp.dot(x, wv_ref[...], preferred_element_type=jnp.float32)


def _win_sc_kernel(qsc_hbm, ksc_hbm, vsc_hbm, attn_hbm, q_t, k_t, v_t):
    cid = jax.lax.axis_index("c")
    sid = jax.lax.axis_index("s")
    wid = cid * NSUB + sid
    row0 = wid * ROWS_W

    pltpu.sync_copy(qsc_hbm.at[pl.ds(row0, ROWS_W)], q_t)

    @pl.when(wid == 0)
    def _stage_first():
        zero = jnp.zeros((LANES,), jnp.float32)

        def zbody(j, carry):
            for c in range(NCHUNK):
                k_t[j, pl.ds(c * LANES, LANES)] = zero
                v_t[j, pl.ds(c * LANES, LANES)] = zero
            return carry
        jax.lax.fori_loop(0, WINDOW, zbody, 0)
        pltpu.sync_copy(ksc_hbm.at[pl.ds(0, ROWS_W)],
                        k_t.at[pl.ds(WINDOW, ROWS_W)])
        pltpu.sync_copy(vsc_hbm.at[pl.ds(0, ROWS_W)],
                        v_t.at[pl.ds(WINDOW, ROWS_W)])

    @pl.when(wid > 0)
    def _stage_interior():
        pltpu.sync_copy(ksc_hbm.at[pl.ds(row0 - WINDOW, KROWS)], k_t)
        pltpu.sync_copy(vsc_hbm.at[pl.ds(row0 - WINDOW, KROWS)], v_t)

    GRP = 4

    def cbody(c, carry):
        col = c * LANES

        def mbody(j8, m):
            j = j8 * 8
            for u in range(8):
                m = jnp.maximum(m, jnp.abs(k_t[j + u, pl.ds(col, LANES)]))
            return m
        mcol = jax.lax.fori_loop(0, KROWS // 8, mbody,
                                 jnp.zeros((LANES,), jnp.float32))

        def gbody(g, carry2):
            s0 = g * GRP
            qs, ms, dens, accs = [], [], [], []
            for ds in range(GRP):
                q16 = q_t[s0 + ds, pl.ds(col, LANES)] * INV_SQRT
                m16 = jnp.abs(q16) * mcol
                ninv = jnp.maximum(WINDOW - 1 - (row0 + s0 + ds),
                                   0).astype(jnp.float32)
                qs.append(q16)
                ms.append(m16)
                dens.append(jnp.exp(-m16) * (-ninv))
                accs.append(jnp.zeros((LANES,), jnp.float32))
            for t in range(WINDOW + GRP - 1):
                kj = k_t[s0 + t + 1, pl.ds(col, LANES)]
                vj = v_t[s0 + t + 1, pl.ds(col, LANES)]
                for ds in range(max(0, t - WINDOW + 1), min(GRP - 1, t) + 1):
                    e = jnp.exp(qs[ds] * kj - ms[ds])
                    dens[ds] = dens[ds] + e
                    accs[ds] = accs[ds] + e * vj
            for ds in range(GRP):
                q_t[s0 + ds, pl.ds(col, LANES)] = accs[ds] / dens[ds]
            return carry2

        return jax.lax.fori_loop(0, ROWS_W // GRP, gbody, carry)

    jax.lax.fori_loop(0, NCHUNK, cbody, 0)

    pltpu.sync_copy(q_t, attn_hbm.at[pl.ds(row0, ROWS_W)])


def _win_tc_kernel(qt_ref, kt_ref, vt_ref, wo_ref, out_ref,
                   slab_k_ref, slab_v_ref, shk, shv, attn_ref):
    i = pl.program_id(0)
    base = pl.multiple_of(i * BLK, BLK)

    @pl.when(i == 0)
    def _first_block_slab():
        slab_k_ref[0:WINDOW, :] = jnp.zeros((WINDOW, C_TC), jnp.float32)
        slab_v_ref[0:WINDOW, :] = jnp.zeros((WINDOW, C_TC), jnp.float32)
        slab_k_ref[pl.ds(WINDOW, BLK), :] = kt_ref[pl.ds(0, BLK), :]
        slab_v_ref[pl.ds(WINDOW, BLK), :] = vt_ref[pl.ds(0, BLK), :]

    @pl.when(i > 0)
    def _interior_slab():
        slab_k_ref[0:SLAB, :] = kt_ref[pl.ds(base - WINDOW, SLAB), :]
        slab_v_ref[0:SLAB, :] = vt_ref[pl.ds(base - WINDOW, SLAB), :]

    slab_k = slab_k_ref[...]
    slab_v = slab_v_ref[...]
    for r in range(8):
        shk[r] = jax.lax.slice(slab_k, (r, 0), (r + SLAB, C_TC))
        shv[r] = jax.lax.slice(slab_v, (r, 0), (r + SLAB, C_TC))

    colmax = jnp.max(
        jnp.abs(jax.lax.slice(slab_k, (0, 0), (SLAB, C_TC))),
        axis=0, keepdims=True)

    TR = 16

    def tile_body(t, carry):
        t8 = pl.multiple_of(t * TR, TR)
        q_scaled = qt_ref[pl.ds(t8, TR), :] * INV_SQRT
        m = jnp.abs(q_scaled) * colmax
        rows = jax.lax.broadcasted_iota(jnp.int32, (TR, C_TC), 0) + base + t8
        n_invalid = jnp.maximum(WINDOW - 1 - rows, 0).astype(jnp.float32)
        den = -n_invalid * jnp.exp(-m)
        acc = jnp.zeros((TR, C_TC), jnp.float32)
        for w in range(WINDOW):
            s0 = w + 1
            r = s0 % 8
            a = pl.multiple_of(s0 - r + t8, TR)
            e = jnp.exp(q_scaled * shk[r, pl.ds(a, TR), :] - m)
            den = den + e
            acc = acc + e * shv[r, pl.ds(a, TR), :]
        attn_ref[pl.ds(t8, TR), :] = acc / den
        return carry

    jax.lax.fori_loop(0, BLK // TR, tile_body, 0)

    out_ref[...] = jnp.dot(attn_ref[...].astype(jnp.bfloat16), wo_ref[...],
                           preferred_element_type=jnp.float32)


def _combine_kernel(attnsc_ref, wosc_ref, part_ref, out_ref):
    out_ref[...] = part_ref[...] + jnp.dot(
        attnsc_ref[...].astype(jnp.bfloat16), wosc_ref[...],
        preferred_element_type=jnp.float32)


def kernel(x, Wq, Wk, Wv, Wo):
    xs = x.reshape(SEQ, D_MODEL).astype(jnp.bfloat16)
    bf = jnp.bfloat16
    wqt, wkt, wvt, wot = Wq.T.astype(bf), Wk.T.astype(bf), Wv.T.astype(bf), Wo.T.astype(bf)
    wot_sc = wot[:C_SC, :]
    wot_tc = wot[C_SC:, :]

    sc_spec = pl.BlockSpec((BLK, C_SC), lambda i: (i, 0))
    tc_spec = pl.BlockSpec((BLK, C_TC), lambda i: (i, 0))

    def proj(wq_cols, wk_cols, wv_cols, width, spec):
        return pl.pallas_call(
            _proj_kernel,
            grid=(NB,),
            in_specs=[
                pl.BlockSpec((BLK, D_MODEL), lambda i: (i, 0)),
                pl.BlockSpec((D_MODEL, width), lambda i: (0, 0)),
                pl.BlockSpec((D_MODEL, width), lambda i: (0, 0)),
                pl.BlockSpec((D_MODEL, width), lambda i: (0, 0)),
            ],
            out_specs=[spec, spec, spec],
            out_shape=[jax.ShapeDtypeStruct((SEQ, width), jnp.float32)] * 3,
        )(xs, wq_cols, wk_cols, wv_cols)

    qsc, ksc, vsc = proj(wqt[:, :C_SC], wkt[:, :C_SC], wvt[:, :C_SC],
                         C_SC, sc_spec)
    qtc, ktc, vtc = proj(wqt[:, C_SC:], wkt[:, C_SC:], wvt[:, C_SC:],
                         C_TC, tc_spec)

    attn_sc = pl.kernel(
        _win_sc_kernel,
        out_type=jax.ShapeDtypeStruct((SEQ, C_SC), jnp.float32),
        mesh=plsc.VectorSubcoreMesh(
            core_axis_name="c", subcore_axis_name="s",
            num_cores=NCORE, num_subcores=NSUB),
        scratch_types=[
            pltpu.VMEM((ROWS_W, C_SC), jnp.float32),
            pltpu.VMEM((KROWS, C_SC), jnp.float32),
            pltpu.VMEM((KROWS, C_SC), jnp.float32),
        ],
    )(qsc, ksc, vsc)

    full_tc = pl.BlockSpec((SEQ, C_TC), lambda i: (0, 0))
    out_part = pl.pallas_call(
        _win_tc_kernel,
        grid=(NB,),
        in_specs=[
            tc_spec,
            full_tc,
            full_tc,
            pl.BlockSpec((C_TC, D_MODEL), lambda i: (0, 0)),
        ],
        out_specs=pl.BlockSpec((BLK, D_MODEL), lambda i: (i, 0)),
        out_shape=jax.ShapeDtypeStruct((SEQ, D_MODEL), jnp.float32),
        scratch_shapes=[
            pltpu.VMEM((SLAB + 8, C_TC), jnp.float32),
            pltpu.VMEM((SLAB + 8, C_TC), jnp.float32),
            pltpu.VMEM((8, SLAB, C_TC), jnp.float32),
            pltpu.VMEM((8, SLAB, C_TC), jnp.float32),
            pltpu.VMEM((BLK, C_TC), jnp.float32),
        ],
    )(qtc, ktc, vtc, wot_tc)

    out = pl.pallas_call(
        _combine_kernel,
        grid=(NB,),
        in_specs=[
            sc_spec,
            pl.BlockSpec((C_SC, D_MODEL), lambda i: (0, 0)),
            pl.BlockSpec((BLK, D_MODEL), lambda i: (i, 0)),
        ],
        out_specs=pl.BlockSpec((BLK, D_MODEL), lambda i: (i, 0)),
        out_shape=jax.ShapeDtypeStruct((SEQ, D_MODEL), jnp.float32),
    )(attn_sc, wot_sc, out_part)

    return out.reshape(1, SEQ, D_MODEL)


# speedup vs baseline: 1.1458x; 1.0097x over previous
import functools
import jax
import jax.numpy as jnp
from jax.experimental import pallas as pl
from jax.experimental.pallas import tpu as pltpu
from jax.experimental.pallas import tpu_sc as plsc

D_MODEL = 768
WINDOW = 64
SEQ = 2048
BLK = 256
NB = SEQ // BLK
SLAB = BLK + WINDOW
INV_SQRT = 1.0 / (64.0 ** 0.5)

C_SC = 256
C_TC = D_MODEL - C_SC

NCORE, NSUB, LANES = 2, 16, 16
NWORK = NCORE * NSUB
ROWS_W = SEQ // NWORK
KROWS = ROWS_W + WINDOW
NCHUNK = C_SC // LANES


def _proj_kernel(x_ref, wq_ref, wk_ref, wv_ref, q_ref, k_ref, v_ref):
    x = x_ref[...]
    q_ref[...] = jnp.dot(x, wq_ref[...], preferred_element_type=jnp.float32)
    k_ref[...] = jnp.dot(x, wk_ref[...], preferred_element_type=jnp.float32)
    v_ref[...] = jnp.dot(x, wv_ref[...], preferred_element_type=jnp.float32)


def _win_sc_kernel(qsc_hbm, ksc_hbm, vsc_hbm, attn_hbm, q_t, k_t, v_t,
                   sem_q, sem_k, sem_v):
    cid = jax.lax.axis_index("c")
    sid = jax.lax.axis_index("s")
    wid = cid * NSUB + sid
    row0 = wid * ROWS_W

    cq = pltpu.make_async_copy(qsc_hbm.at[pl.ds(row0, ROWS_W)], q_t, sem_q)
    cq.start()

    @pl.when(wid == 0)
    def _stage_first():
        ck = pltpu.make_async_copy(ksc_hbm.at[pl.ds(0, ROWS_W)],
                                   k_t.at[pl.ds(WINDOW, ROWS_W)], sem_k)
        cv = pltpu.make_async_copy(vsc_hbm.at[pl.ds(0, ROWS_W)],
                                   v_t.at[pl.ds(WINDOW, ROWS_W)], sem_v)
        ck.start()
        cv.start()
        zero = jnp.zeros((LANES,), jnp.float32)

        def zbody(j, carry):
            for c in range(NCHUNK):
                k_t[j, pl.ds(c * LANES, LANES)] = zero
                v_t[j, pl.ds(c * LANES, LANES)] = zero
            return carry
        jax.lax.fori_loop(0, WINDOW, zbody, 0)
        ck.wait()
        cv.wait()

    @pl.when(wid > 0)
    def _stage_interior():
        ck = pltpu.make_async_copy(ksc_hbm.at[pl.ds(row0 - WINDOW, KROWS)],
                                   k_t, sem_k)
        cv = pltpu.make_async_copy(vsc_hbm.at[pl.ds(row0 - WINDOW, KROWS)],
                                   v_t, sem_v)
        ck.start()
        cv.start()
        ck.wait()
        cv.wait()

    cq.wait()

    GRP = 4

    def cbody(c, carry):
        col = c * LANES

        def mbody(j8, m):
            j = j8 * 8
            for u in range(8):
                m = jnp.maximum(m, jnp.abs(k_t[j + u, pl.ds(col, LANES)]))
            return m
        mcol = jax.lax.fori_loop(0, KROWS // 8, mbody,
                                 jnp.zeros((LANES,), jnp.float32))

        def gbody(g, carry2):
            s0 = g * GRP
            qs, ms, dens, accs = [], [], [], []
            for ds in range(GRP):
                q16 = q_t[s0 + ds, pl.ds(col, LANES)] * INV_SQRT
                m16 = jnp.abs(q16) * mcol
                ninv = jnp.maximum(WINDOW - 1 - (row0 + s0 + ds),
                                   0).astype(jnp.float32)
                qs.append(q16)
                ms.append(m16)
                dens.append(jnp.exp(-m16) * (-ninv))
                accs.append(jnp.zeros((LANES,), jnp.float32))
            for t in range(WINDOW + GRP - 1):
                kj = k_t[s0 + t + 1, pl.ds(col, LANES)]
                vj = v_t[s0 + t + 1, pl.ds(col, LANES)]
                for ds in range(max(0, t - WINDOW + 1), min(GRP - 1, t) + 1):
                    e = jnp.exp(qs[ds] * kj - ms[ds])
                    dens[ds] = dens[ds] + e
                    accs[ds] = accs[ds] + e * vj
            for ds in range(GRP):
                q_t[s0 + ds, pl.ds(col, LANES)] = accs[ds] / dens[ds]
            return carry2

        return jax.lax.fori_loop(0, ROWS_W // GRP, gbody, carry)

    jax.lax.fori_loop(0, NCHUNK, cbody, 0)

    pltpu.sync_copy(q_t, attn_hbm.at[pl.ds(row0, ROWS_W)])


def _win_tc_kernel(qt_ref, kt_ref, vt_ref, wo_ref, out_ref,
                   slab_k_ref, slab_v_ref, shk, shv, attn_ref):
    i = pl.program_id(0)
    base = pl.multiple_of(i * BLK, BLK)

    @pl.when(i == 0)
    def _first_block_slab():
        slab_k_ref[0:WINDOW, :] = jnp.zeros((WINDOW, C_TC), jnp.float32)
        slab_v_ref[0:WINDOW, :] = jnp.zeros((WINDOW, C_TC), jnp.float32)
        slab_k_ref[pl.ds(WINDOW, BLK), :] = kt_ref[pl.ds(0, BLK), :]
        slab_v_ref[pl.ds(WINDOW, BLK), :] = vt_ref[pl.ds(0, BLK), :]

    @pl.when(i > 0)
    def _interior_slab():
        slab_k_ref[0:SLAB, :] = kt_ref[pl.ds(base - WINDOW, SLAB), :]
        slab_v_ref[0:SLAB, :] = vt_ref[pl.ds(base - WINDOW, SLAB), :]

    slab_k = slab_k_ref[...]
    slab_v = slab_v_ref[...]
    for r in range(8):
        shk[r] = jax.lax.slice(slab_k, (r, 0), (r + SLAB, C_TC))
        shv[r] = jax.lax.slice(slab_v, (r, 0), (r + SLAB, C_TC))

    colmax = jnp.max(
        jnp.abs(jax.lax.slice(slab_k, (0, 0), (SLAB, C_TC))),
        axis=0, keepdims=True)

    TR = 16

    def tile_body(t, carry):
        t8 = pl.multiple_of(t * TR, TR)
        q_scaled = qt_ref[pl.ds(t8, TR), :] * INV_SQRT
        m = jnp.abs(q_scaled) * colmax
        rows = jax.lax.broadcasted_iota(jnp.int32, (TR, C_TC), 0) + base + t8
        n_invalid = jnp.maximum(WINDOW - 1 - rows, 0).astype(jnp.float32)
        den = -n_invalid * jnp.exp(-m)
        acc = jnp.zeros((TR, C_TC), jnp.float32)
        for w in range(WINDOW):
            s0 = w + 1
            r = s0 % 8
            a = pl.multiple_of(s0 - r + t8, TR)
            e = jnp.exp(q_scaled * shk[r, pl.ds(a, TR), :] - m)
            den = den + e
            acc = acc + e * shv[r, pl.ds(a, TR), :]
        attn_ref[pl.ds(t8, TR), :] = acc / den
        return carry

    jax.lax.fori_loop(0, BLK // TR, tile_body, 0)

    out_ref[...] = jnp.dot(attn_ref[...].astype(jnp.bfloat16), wo_ref[...],
                           preferred_element_type=jnp.float32)


def _combine_kernel(attnsc_ref, wosc_ref, part_ref, out_ref):
    out_ref[...] = part_ref[...] + jnp.dot(
        attnsc_ref[...].astype(jnp.bfloat16), wosc_ref[...],
        preferred_element_type=jnp.float32)


def kernel(x, Wq, Wk, Wv, Wo):
    xs = x.reshape(SEQ, D_MODEL).astype(jnp.bfloat16)
    bf = jnp.bfloat16
    wqt, wkt, wvt, wot = Wq.T.astype(bf), Wk.T.astype(bf), Wv.T.astype(bf), Wo.T.astype(bf)
    wot_sc = wot[:C_SC, :]
    wot_tc = wot[C_SC:, :]

    sc_spec = pl.BlockSpec((BLK, C_SC), lambda i: (i, 0))
    tc_spec = pl.BlockSpec((BLK, C_TC), lambda i: (i, 0))

    def proj(wq_cols, wk_cols, wv_cols, width, spec):
        return pl.pallas_call(
            _proj_kernel,
            grid=(NB,),
            in_specs=[
                pl.BlockSpec((BLK, D_MODEL), lambda i: (i, 0)),
                pl.BlockSpec((D_MODEL, width), lambda i: (0, 0)),
                pl.BlockSpec((D_MODEL, width), lambda i: (0, 0)),
                pl.BlockSpec((D_MODEL, width), lambda i: (0, 0)),
            ],
            out_specs=[spec, spec, spec],
            out_shape=[jax.ShapeDtypeStruct((SEQ, width), jnp.float32)] * 3,
        )(xs, wq_cols, wk_cols, wv_cols)

    qsc, ksc, vsc = proj(wqt[:, :C_SC], wkt[:, :C_SC], wvt[:, :C_SC],
                         C_SC, sc_spec)
    qtc, ktc, vtc = proj(wqt[:, C_SC:], wkt[:, C_SC:], wvt[:, C_SC:],
                         C_TC, tc_spec)

    attn_sc = pl.kernel(
        _win_sc_kernel,
        out_type=jax.ShapeDtypeStruct((SEQ, C_SC), jnp.float32),
        mesh=plsc.VectorSubcoreMesh(
            core_axis_name="c", subcore_axis_name="s",
            num_cores=NCORE, num_subcores=NSUB),
        scratch_types=[
            pltpu.VMEM((ROWS_W, C_SC), jnp.float32),
            pltpu.VMEM((KROWS, C_SC), jnp.float32),
            pltpu.VMEM((KROWS, C_SC), jnp.float32),
            pltpu.SemaphoreType.DMA,
            pltpu.SemaphoreType.DMA,
            pltpu.SemaphoreType.DMA,
        ],
    )(qsc, ksc, vsc)

    full_tc = pl.BlockSpec((SEQ, C_TC), lambda i: (0, 0))
    out_part = pl.pallas_call(
        _win_tc_kernel,
        grid=(NB,),
        in_specs=[
            tc_spec,
            full_tc,
            full_tc,
            pl.BlockSpec((C_TC, D_MODEL), lambda i: (0, 0)),
        ],
        out_specs=pl.BlockSpec((BLK, D_MODEL), lambda i: (i, 0)),
        out_shape=jax.ShapeDtypeStruct((SEQ, D_MODEL), jnp.float32),
        scratch_shapes=[
            pltpu.VMEM((SLAB + 8, C_TC), jnp.float32),
            pltpu.VMEM((SLAB + 8, C_TC), jnp.float32),
            pltpu.VMEM((8, SLAB, C_TC), jnp.float32),
            pltpu.VMEM((8, SLAB, C_TC), jnp.float32),
            pltpu.VMEM((BLK, C_TC), jnp.float32),
        ],
    )(qtc, ktc, vtc, wot_tc)

    out = pl.pallas_call(
        _combine_kernel,
        grid=(NB,),
        in_specs=[
            sc_spec,
            pl.BlockSpec((C_SC, D_MODEL), lambda i: (0, 0)),
            pl.BlockSpec((BLK, D_MODEL), lambda i: (i, 0)),
        ],
        out_specs=pl.BlockSpec((BLK, D_MODEL), lambda i: (i, 0)),
        out_shape=jax.ShapeDtypeStruct((SEQ, D_MODEL), jnp.float32),
    )(attn_sc, wot_sc, out_part)

    return out.reshape(1, SEQ, D_MODEL)
